# baseline (speedup 1.0000x reference)
import functools
import numpy as np
import jax
import jax.numpy as jnp
from jax import lax
from jax.experimental import pallas as pl
from jax.experimental.pallas import tpu as pltpu
from jax.experimental.pallas import tpu_sc as plsc

_EXTRA_IDXS = np.array([
    9120, 9929, 9448, 616, 6,
    5770, 5780, 8846, 8463, 8474, 8635,
    5361, 4933, 5058, 5169, 5286,
    8079, 7669, 7794, 7905, 8022
], dtype=np.int32)

_B, _V, _C = 1024, 10475, 3
_J, _E = 55, 21
_NW = 32
_BPW = _B // _NW
_PPW = _BPW * _E
_NROW = 2 * _PPW
_CH = 112
_NCH = _NROW // _CH
_OW = (_J + _E) * _C
_JW = _J * _C
_NT = _PPW * _C // 16
_T8 = _B * _V * _C // 8


def _build_index_tables():
    b = np.arange(_B, dtype=np.int64)[:, None]
    e = (b * _V + _EXTRA_IDXS[None, :]) * _C
    e = e.reshape(_NW, _PPW)
    k = e // 8
    rid = np.stack([k, k + 1], axis=-1).reshape(_NW, _NCH, _CH).astype(np.int32)

    g = np.arange(_PPW * _C, dtype=np.int64)
    p, c = g // _C, g % _C
    word = p * 16 + (e[:, p] % 8) + c[None, :]
    gr, gc = (word // 8).astype(np.int32), (word % 8).astype(np.int32)
    sb = np.broadcast_to((p // _E).astype(np.int32), word.shape)
    sj = np.broadcast_to((_J + (p % _E)).astype(np.int32), word.shape)
    sc = np.broadcast_to(c.astype(np.int32), word.shape)
    idxv = np.stack([gr, gc, sb, sj, sc], axis=1)
    idxv = idxv.reshape(_NW, 5, _NT, 16).transpose(0, 2, 1, 3)
    return rid, np.ascontiguousarray(idxv)


_RID, _IDXV = _build_index_tables()


def _sc_body(tab8, joints2, rid, idxv, out, rid_v, idxv_v, blk, wbuf, sem):
    wid = lax.axis_index("s") * 2 + lax.axis_index("c")
    b0 = wid * _BPW
    pltpu.sync_copy(rid.at[wid], rid_v)
    pltpu.sync_copy(idxv.at[wid], idxv_v)
    copies = [
        pltpu.async_copy(tab8.at[rid_v.at[ch]], wbuf.at[pl.ds(ch * _CH, _CH)], sem)
        for ch in range(_NCH)
    ]
    pltpu.sync_copy(joints2.at[pl.ds(b0, _BPW)], blk.at[:, pl.ds(0, _J)])
    for cp in copies:
        cp.wait()
    for t in range(_NT):
        x = plsc.load_gather(wbuf, [idxv_v[t, 0], idxv_v[t, 1]])
        plsc.store_scatter(blk, [idxv_v[t, 2], idxv_v[t, 3], idxv_v[t, 4]], x)
    pltpu.sync_copy(blk, out.at[pl.ds(b0, _BPW)])


@jax.jit
def kernel(vertices, joints):
    tab8 = vertices.reshape(_T8, 8)
    mesh = plsc.VectorSubcoreMesh(core_axis_name="c", subcore_axis_name="s")
    run = pl.kernel(
        _sc_body,
        out_type=jax.ShapeDtypeStruct((_B, _J + _E, _C), jnp.float32),
        mesh=mesh,
        scratch_types=[
            pltpu.VMEM((_NCH, _CH), jnp.int32),
            pltpu.VMEM((_NT, 5, 16), jnp.int32),
            pltpu.VMEM((_BPW, _J + _E, _C), jnp.float32),
            pltpu.VMEM((_NROW, 8), jnp.float32),
            pltpu.SemaphoreType.DMA,
        ],
        compiler_params=pltpu.CompilerParams(
            use_tc_tiling_on_sc=False, needs_layout_passes=False),
    )
    return run(tab8, joints, jnp.asarray(_RID), jnp.asarray(_IDXV))


# speedup vs baseline: 10622.8117x; 10622.8117x over previous
import numpy as np
import jax
import jax.numpy as jnp
from jax.experimental import pallas as pl

_EXTRA_IDXS = np.array([
    9120, 9929, 9448, 616, 6,
    5770, 5780, 8846, 8463, 8474, 8635,
    5361, 4933, 5058, 5169, 5286,
    8079, 7669, 7794, 7905, 8022
], dtype=np.int32)

_B, _V, _C = 1024, 10475, 3
_J, _E = 55, 21


def _body(*refs):
    jt_ref = refs[0]
    vrefs = refs[1:1 + _E]
    out_ref = refs[1 + _E]
    out_ref[0, 0:_J, :] = jt_ref[0]
    for j in range(_E):
        s = int(_EXTRA_IDXS[j]) % 8
        out_ref[0, _J + j:_J + j + 1, :] = vrefs[j][0, s:s + 1, :]


@jax.jit
def kernel(vertices, joints):
    vt = jnp.transpose(vertices, (2, 1, 0))
    jt = jnp.transpose(joints, (2, 1, 0))
    in_specs = [pl.BlockSpec((1, _J, _B), lambda c: (c, 0, 0))]
    for j in range(_E):
        blk = int(_EXTRA_IDXS[j]) // 8
        in_specs.append(
            pl.BlockSpec((1, 8, _B), lambda c, _blk=blk: (c, _blk, 0)))
    out_t = pl.pallas_call(
        _body,
        grid=(_C,),
        in_specs=in_specs,
        out_specs=pl.BlockSpec((1, _J + _E, _B), lambda c: (c, 0, 0)),
        out_shape=jax.ShapeDtypeStruct((_C, _J + _E, _B), jnp.float32),
    )(jt, *([vt] * _E))
    return jnp.transpose(out_t, (2, 1, 0))


# speedup vs baseline: 13689.3432x; 1.2887x over previous
import numpy as np
import jax
import jax.numpy as jnp
from jax.experimental import pallas as pl

_EXTRA_IDXS = np.array([
    9120, 9929, 9448, 616, 6,
    5770, 5780, 8846, 8463, 8474, 8635,
    5361, 4933, 5058, 5169, 5286,
    8079, 7669, 7794, 7905, 8022
], dtype=np.int32)

_B, _V, _C = 1024, 10475, 3
_J, _E = 55, 21


def _body(*refs):
    jt_ref = refs[0]
    vrefs = refs[1:1 + _E]
    out_ref = refs[1 + _E]
    out_ref[:, 0:_J, :] = jt_ref[:]
    for j in range(_E):
        s = int(_EXTRA_IDXS[j]) % 8
        out_ref[:, _J + j:_J + j + 1, :] = vrefs[j][:, s:s + 1, :]


@jax.jit
def kernel(vertices, joints):
    vt = jnp.transpose(vertices, (2, 1, 0))
    jt = jnp.transpose(joints, (2, 1, 0))
    in_specs = [pl.BlockSpec((_C, _J, _B), lambda i: (0, 0, 0))]
    for j in range(_E):
        blk = int(_EXTRA_IDXS[j]) // 8
        in_specs.append(
            pl.BlockSpec((_C, 8, _B), lambda i, _blk=blk: (0, _blk, 0)))
    out_t = pl.pallas_call(
        _body,
        grid=(1,),
        in_specs=in_specs,
        out_specs=pl.BlockSpec((_C, _J + _E, _B), lambda i: (0, 0, 0)),
        out_shape=jax.ShapeDtypeStruct((_C, _J + _E, _B), jnp.float32),
    )(jt, *([vt] * _E))
    return jnp.transpose(out_t, (2, 1, 0))
